# baseline (speedup 1.0000x reference)
import functools

import jax
import jax.numpy as jnp
from jax import lax
from jax.experimental import pallas as pl
from jax.experimental.pallas import tpu as pltpu
from jax.experimental.pallas import tpu_sc as plsc

DIM = 64
PDIM = 128
N_HEADS = 26
BATCH = 16384
N_ROWS = BATCH * N_HEADS
N_TABLE = 2600000
NC, NS, L = 2, 16, 16
NW = NC * NS
ROWS_PER_W = N_ROWS // NW
CHUNK = 128
N_CHUNKS = ROWS_PER_W // CHUNK
VREGS_PER_CHUNK = CHUNK // L
NBUF = 4
N_GROUPS = N_CHUNKS // NBUF
TBLK = 512


def _tc_transpose_pad(tbl_t_ref, out_ref):
    x = tbl_t_ref[...]
    out_ref[:, 0:DIM] = x.T


def _sc_gather(ids_hbm, offs_hbm, table_hbm, out_hbm,
               idx_v, offs_v, rows_v, gsem, osem):
    wid = lax.axis_index("s") * NC + lax.axis_index("c")
    pltpu.sync_copy(ids_hbm.at[wid], idx_v)
    pltpu.sync_copy(offs_hbm, offs_v)
    out_base = wid * ROWS_PER_W

    def add_offsets(j):
        for k in range(VREGS_PER_CHUNK):
            sl = pl.ds(k * L, L)
            idx_v[j, sl] = idx_v[j, sl] + offs_v[j, sl]

    def gather(j, b):
        return pltpu.make_async_copy(
            table_hbm.at[idx_v.at[j]], rows_v.at[b], gsem.at[b])

    def writeback(j, b):
        start = pl.multiple_of(out_base + j * CHUNK, CHUNK)
        return pltpu.make_async_copy(
            rows_v.at[b],
            out_hbm.at[pl.ds(start, CHUNK)],
            osem.at[b])

    for b in range(NBUF):
        add_offsets(b)
        gather(b, b).start()

    def group_body(g, carry):
        for b in range(NBUF):
            j = g * NBUF + b
            gather(j, b).wait()
            writeback(j, b).start()
            jn = j + NBUF
            add_offsets(jn)
            writeback(j, b).wait()
            gather(jn, b).start()
        return carry

    lax.fori_loop(0, N_GROUPS - 1, group_body, 0)

    for b in range(NBUF):
        j = (N_GROUPS - 1) * NBUF + b
        gather(j, b).wait()
        writeback(j, b).start()
    for b in range(NBUF):
        j = (N_GROUPS - 1) * NBUF + b
        writeback(j, b).wait()


@jax.jit
def _run(ids, offs_tiled, table_t):
    n_blocks = (N_TABLE + TBLK - 1) // TBLK
    tpad = pl.pallas_call(
        _tc_transpose_pad,
        grid=(n_blocks,),
        in_specs=[pl.BlockSpec((DIM, TBLK), lambda i: (0, i))],
        out_specs=pl.BlockSpec((TBLK, PDIM), lambda i: (i, 0)),
        out_shape=jax.ShapeDtypeStruct((N_TABLE, PDIM), jnp.float32),
    )(table_t)

    mesh = plsc.VectorSubcoreMesh(core_axis_name="c", subcore_axis_name="s")
    f = pl.kernel(
        _sc_gather,
        out_type=jax.ShapeDtypeStruct((N_ROWS, PDIM), jnp.float32),
        mesh=mesh,
        scratch_types=[
            pltpu.VMEM((N_CHUNKS, CHUNK), jnp.int32),
            pltpu.VMEM((N_CHUNKS, CHUNK), jnp.int32),
            pltpu.VMEM((NBUF, CHUNK, PDIM), jnp.float32),
            pltpu.SemaphoreType.DMA((NBUF,)),
            pltpu.SemaphoreType.DMA((NBUF,)),
        ],
    )
    return f(ids, offs_tiled, tpad)


def kernel(input_ids, offsets, table):
    ids = input_ids.astype(jnp.int32).reshape(NW, N_CHUNKS, CHUNK)
    offs_tiled = jnp.tile(offsets.astype(jnp.int32),
                          ROWS_PER_W // N_HEADS).reshape(N_CHUNKS, CHUNK)
    table_t = table.astype(jnp.float32).T
    out128 = _run(ids, offs_tiled, table_t)
    return out128[:, :DIM].reshape(BATCH, N_HEADS, DIM)


# speedup vs baseline: 1.4769x; 1.4769x over previous
import functools

import jax
import jax.numpy as jnp
from jax import lax
from jax.experimental import pallas as pl
from jax.experimental.pallas import tpu as pltpu
from jax.experimental.pallas import tpu_sc as plsc

DIM = 64
PDIM = 128
N_HEADS = 26
BATCH = 16384
N_ROWS = BATCH * N_HEADS
N_TABLE = 2600000
NC, NS, L = 2, 16, 16
NW = NC * NS
ROWS_PER_W = N_ROWS // NW
CHUNK = 128
N_CHUNKS = ROWS_PER_W // CHUNK
VREGS_PER_CHUNK = CHUNK // L
NBUF = 4
N_GROUPS = N_CHUNKS // NBUF
TBLK = 1024


def _tc_transpose_pad(tbl_t_ref, out_ref):
    x = tbl_t_ref[...]
    ident = jnp.eye(DIM, dtype=jnp.float32)
    xt = lax.dot_general(x, ident, (((0,), (0,)), ((), ())),
                         preferred_element_type=jnp.float32)
    out_ref[:, 0:DIM] = xt


def _sc_gather(ids_hbm, offs_hbm, table_hbm, out_hbm,
               idx_v, offs_v, rows_v, gsem, osem):
    wid = lax.axis_index("s") * NC + lax.axis_index("c")
    pltpu.sync_copy(ids_hbm.at[wid], idx_v)
    pltpu.sync_copy(offs_hbm, offs_v)
    out_base = wid * ROWS_PER_W

    def add_offsets(j):
        for k in range(VREGS_PER_CHUNK):
            sl = pl.ds(k * L, L)
            idx_v[j, sl] = idx_v[j, sl] + offs_v[j, sl]

    def gather(j, b):
        return pltpu.make_async_copy(
            table_hbm.at[idx_v.at[j]], rows_v.at[b], gsem.at[b])

    def writeback(j, b):
        start = pl.multiple_of(out_base + j * CHUNK, CHUNK)
        return pltpu.make_async_copy(
            rows_v.at[b],
            out_hbm.at[pl.ds(start, CHUNK)],
            osem.at[b])

    for b in range(NBUF):
        add_offsets(b)
        gather(b, b).start()

    def group_body(g, carry):
        for b in range(NBUF):
            j = g * NBUF + b
            gather(j, b).wait()
            writeback(j, b).start()
            jn = j + NBUF
            add_offsets(jn)
            writeback(j, b).wait()
            gather(jn, b).start()
        return carry

    lax.fori_loop(0, N_GROUPS - 1, group_body, 0)

    for b in range(NBUF):
        j = (N_GROUPS - 1) * NBUF + b
        gather(j, b).wait()
        writeback(j, b).start()
    for b in range(NBUF):
        j = (N_GROUPS - 1) * NBUF + b
        writeback(j, b).wait()


@jax.jit
def _run(ids, offs_tiled, table_t):
    n_blocks = (N_TABLE + TBLK - 1) // TBLK
    tpad = pl.pallas_call(
        _tc_transpose_pad,
        grid=(n_blocks,),
        in_specs=[pl.BlockSpec((DIM, TBLK), lambda i: (0, i))],
        out_specs=pl.BlockSpec((TBLK, PDIM), lambda i: (i, 0)),
        out_shape=jax.ShapeDtypeStruct((N_TABLE, PDIM), jnp.float32),
    )(table_t)

    mesh = plsc.VectorSubcoreMesh(core_axis_name="c", subcore_axis_name="s")
    f = pl.kernel(
        _sc_gather,
        out_type=jax.ShapeDtypeStruct((N_ROWS, PDIM), jnp.float32),
        mesh=mesh,
        scratch_types=[
            pltpu.VMEM((N_CHUNKS, CHUNK), jnp.int32),
            pltpu.VMEM((N_CHUNKS, CHUNK), jnp.int32),
            pltpu.VMEM((NBUF, CHUNK, PDIM), jnp.float32),
            pltpu.SemaphoreType.DMA((NBUF,)),
            pltpu.SemaphoreType.DMA((NBUF,)),
        ],
    )
    return f(ids, offs_tiled, tpad)


def kernel(input_ids, offsets, table):
    ids = input_ids.astype(jnp.int32).reshape(NW, N_CHUNKS, CHUNK)
    offs_tiled = jnp.tile(offsets.astype(jnp.int32),
                          ROWS_PER_W // N_HEADS).reshape(N_CHUNKS, CHUNK)
    table_t = table.astype(jnp.float32).T
    out128 = _run(ids, offs_tiled, table_t)
    return out128[:, :DIM].reshape(BATCH, N_HEADS, DIM)


# speedup vs baseline: 1.9590x; 1.3264x over previous
import functools

import jax
import jax.numpy as jnp
from jax import lax
from jax.experimental import pallas as pl
from jax.experimental.pallas import tpu as pltpu
from jax.experimental.pallas import tpu_sc as plsc

DIM = 64
PDIM = 128
N_HEADS = 26
BATCH = 16384
N_ROWS = BATCH * N_HEADS
NC, NS, L = 2, 16, 16
NW = NC * NS
ROWS_PER_W = N_ROWS // NW
CHUNK = 128
N_CHUNKS = ROWS_PER_W // CHUNK
VREGS_PER_CHUNK = CHUNK // L
NBUF = 4
N_GROUPS = N_CHUNKS // NBUF


def _sc_gather(ids_hbm, offs_hbm, table_hbm, out_hbm,
               idx_v, offs_v, rows_v, gsem, osem):
    wid = lax.axis_index("s") * NC + lax.axis_index("c")
    pltpu.sync_copy(ids_hbm.at[wid], idx_v)
    pltpu.sync_copy(offs_hbm, offs_v)
    out_base = wid * ROWS_PER_W

    def add_offsets(j):
        for k in range(VREGS_PER_CHUNK):
            sl = pl.ds(k * L, L)
            idx_v[j, sl] = idx_v[j, sl] + offs_v[j, sl]

    def gather(j, b):
        return pltpu.make_async_copy(
            table_hbm.at[idx_v.at[j]], rows_v.at[b], gsem.at[b])

    def writeback(j, b):
        start = pl.multiple_of(out_base + j * CHUNK, CHUNK)
        return pltpu.make_async_copy(
            rows_v.at[b],
            out_hbm.at[pl.ds(start, CHUNK)],
            osem.at[b])

    for b in range(NBUF):
        add_offsets(b)
        gather(b, b).start()

    def group_body(g, carry):
        for b in range(NBUF):
            j = g * NBUF + b
            gather(j, b).wait()
            writeback(j, b).start()
            jn = j + NBUF
            add_offsets(jn)
            writeback(j, b).wait()
            gather(jn, b).start()
        return carry

    lax.fori_loop(0, N_GROUPS - 1, group_body, 0)

    for b in range(NBUF):
        j = (N_GROUPS - 1) * NBUF + b
        gather(j, b).wait()
        writeback(j, b).start()
    for b in range(NBUF):
        j = (N_GROUPS - 1) * NBUF + b
        writeback(j, b).wait()


@jax.jit
def _run(ids, offs_tiled, table_padded):
    mesh = plsc.VectorSubcoreMesh(core_axis_name="c", subcore_axis_name="s")
    f = pl.kernel(
        _sc_gather,
        out_type=jax.ShapeDtypeStruct((N_ROWS, PDIM), jnp.float32),
        mesh=mesh,
        scratch_types=[
            pltpu.VMEM((N_CHUNKS, CHUNK), jnp.int32),
            pltpu.VMEM((N_CHUNKS, CHUNK), jnp.int32),
            pltpu.VMEM((NBUF, CHUNK, PDIM), jnp.float32),
            pltpu.SemaphoreType.DMA((NBUF,)),
            pltpu.SemaphoreType.DMA((NBUF,)),
        ],
    )
    return f(ids, offs_tiled, table_padded)


def kernel(input_ids, offsets, table):
    ids = input_ids.astype(jnp.int32).reshape(NW, N_CHUNKS, CHUNK)
    offs_tiled = jnp.tile(offsets.astype(jnp.int32),
                          ROWS_PER_W // N_HEADS).reshape(N_CHUNKS, CHUNK)
    tpad = jnp.pad(table.astype(jnp.float32), ((0, 0), (0, PDIM - DIM)))
    out = _run(ids, offs_tiled, tpad)
    return out[:, :DIM].reshape(BATCH, N_HEADS, DIM)
